# Initial kernel scaffold; baseline (speedup 1.0000x reference)
import jax
import jax.numpy as jnp
from jax.experimental import pallas as pl


def kernel(x, edge_index, feature_importance, W1, b1, gamma, beta, running_mean, running_var, W2, b2):
    raise NotImplementedError("write your pallas kernel here")


# speedup vs baseline: 16.6942x; 16.6942x over previous
import functools

import jax
import jax.numpy as jnp
from jax import lax
from jax.experimental import pallas as pl
from jax.experimental.pallas import tpu as pltpu
from jax.experimental.pallas import tpu_sc as plsc

N = 10000
E = 320000
D = 128
OUT = 2

NC = 2
NS = 16
L = 16
NW = NC * NS
NPAD = 10240
E_PAD = 327680
E_PW = E_PAD // NW
CH = 128
NCHUNK = E_PW // CH
ROWS_PER_SUB = NPAD // NS

_mesh = plsc.VectorSubcoreMesh(core_axis_name="c", subcore_axis_name="s")


def _make_sc_scatter(width):
  @functools.partial(
      pl.kernel,
      out_type=jax.ShapeDtypeStruct((NC, NPAD, width), jnp.float32),
      mesh=_mesh,
      compiler_params=pltpu.CompilerParams(use_tc_tiling_on_sc=(width == D)),
      scratch_types=[
          pltpu.VMEM((CH,), jnp.int32),
          pltpu.VMEM((CH,), jnp.int32),
          pltpu.VMEM((CH, width), jnp.float32),
          pltpu.VMEM((CH, width), jnp.float32),
          pltpu.VMEM_SHARED((NPAD, width), jnp.float32),
          pltpu.SemaphoreType.DMA,
      ],
  )
  def k(table_hbm, src_hbm, dst_hbm, out_hbm, src_v, dst_v, rows_v, z_v,
        acc, sem):
    c = lax.axis_index("c")
    s = lax.axis_index("s")
    w = c * NS + s

    @pl.loop(0, CH)
    def _(i):
      @pl.loop(0, width, step=L)
      def _(j):
        z_v[i, pl.ds(j, L)] = jnp.zeros((L,), jnp.float32)

    @pl.loop(0, ROWS_PER_SUB, step=CH)
    def _(r):
      pltpu.sync_copy(z_v, acc.at[pl.ds(s * ROWS_PER_SUB + r, CH)])

    plsc.subcore_barrier()

    @pl.loop(0, NCHUNK)
    def _(j):
      pltpu.sync_copy(src_hbm.at[w, j], src_v)
      pltpu.sync_copy(dst_hbm.at[w, j], dst_v)
      pltpu.async_copy(table_hbm.at[src_v], rows_v, sem).wait()
      pltpu.sync_copy(rows_v, acc.at[dst_v], add=True)

    plsc.subcore_barrier()
    pltpu.sync_copy(acc.at[pl.ds(s * ROWS_PER_SUB, ROWS_PER_SUB)],
                    out_hbm.at[c].at[pl.ds(s * ROWS_PER_SUB, ROWS_PER_SUB)])

  return k


_sc_scatter128 = _make_sc_scatter(D)
_sc_scatter16 = _make_sc_scatter(16)


@functools.partial(
    pl.kernel,
    out_type=jax.ShapeDtypeStruct((NC, NPAD, 16), jnp.float32),
    mesh=_mesh,
    scratch_types=[
        pltpu.VMEM((CH,), jnp.int32),
        pltpu.VMEM((CH, 16), jnp.float32),
        pltpu.VMEM((CH, 16), jnp.float32),
        pltpu.VMEM_SHARED((NPAD, 16), jnp.float32),
    ],
)
def _sc_degree(dst_hbm, out_hbm, dst_v, one_v, z_v, acc):
  c = lax.axis_index("c")
  s = lax.axis_index("s")
  w = c * NS + s

  @pl.loop(0, CH)
  def _(i):
    one_v[i, :] = jnp.ones((16,), jnp.float32)
    z_v[i, :] = jnp.zeros((16,), jnp.float32)

  @pl.loop(0, ROWS_PER_SUB, step=CH)
  def _(r):
    pltpu.sync_copy(z_v, acc.at[pl.ds(s * ROWS_PER_SUB + r, CH)])

  plsc.subcore_barrier()

  @pl.loop(0, NCHUNK)
  def _(j):
    pltpu.sync_copy(dst_hbm.at[w, j], dst_v)
    pltpu.sync_copy(one_v, acc.at[dst_v], add=True)

  plsc.subcore_barrier()
  pltpu.sync_copy(acc.at[pl.ds(s * ROWS_PER_SUB, ROWS_PER_SUB)],
                  out_hbm.at[c].at[pl.ds(s * ROWS_PER_SUB, ROWS_PER_SUB)])


def _tc_prep_body(x_ref, fi_ref, w1_ref, d_ref, g_ref):
  fw = jax.nn.sigmoid(fi_ref[:])
  xw = x_ref[:] * fw[None, :]
  h0 = jnp.dot(xw, w1_ref[:], preferred_element_type=jnp.float32)
  deg = d_ref[0][:, 0:1] + d_ref[1][:, 0:1] + 1.0
  dis = lax.rsqrt(deg)
  g_ref[:] = h0 * dis


def _tc_mid_body(p_ref, g_ref, d_ref, b1_ref, ga_ref, be_ref, rm_ref, rv_ref,
                 w2_ref, q_ref):
  deg = d_ref[0][:, 0:1] + d_ref[1][:, 0:1] + 1.0
  dis = lax.rsqrt(deg)
  h1 = (p_ref[0] + p_ref[1] + g_ref[:]) * dis + b1_ref[:][None, :]
  inv = lax.rsqrt(rv_ref[:] + 1e-5)
  h = (h1 - rm_ref[:][None, :]) * (inv * ga_ref[:])[None, :] + be_ref[:][None, :]
  h = jnp.maximum(h, 0.0)
  q_ref[:] = jnp.dot(h, w2_ref[:], preferred_element_type=jnp.float32) * dis


def _tc_final_body(s_ref, q_ref, d_ref, b2_ref, o_ref):
  deg = d_ref[0][:, 0:1] + d_ref[1][:, 0:1] + 1.0
  dis = lax.rsqrt(deg)
  o_ref[:] = (s_ref[0] + s_ref[1] + q_ref[:]) * dis + b2_ref[:][None, :]


_tc_prep = pl.pallas_call(
    _tc_prep_body, out_shape=jax.ShapeDtypeStruct((NPAD, D), jnp.float32))
_tc_mid = pl.pallas_call(
    _tc_mid_body, out_shape=jax.ShapeDtypeStruct((NPAD, 16), jnp.float32))
_tc_final = pl.pallas_call(
    _tc_final_body, out_shape=jax.ShapeDtypeStruct((NPAD, 16), jnp.float32))


@jax.jit
def kernel(x, edge_index, feature_importance, W1, b1, gamma, beta,
           running_mean, running_var, W2, b2):
  src = edge_index[0].astype(jnp.int32)
  dst = edge_index[1].astype(jnp.int32)
  npadedge = E_PAD - E
  pad_idx = (N + 224 + (jnp.arange(npadedge, dtype=jnp.int32) % 16))
  src_p = jnp.concatenate([src, pad_idx]).reshape(NW, NCHUNK, CH)
  dst_p = jnp.concatenate([dst, pad_idx]).reshape(NW, NCHUNK, CH)
  x_p = jnp.pad(x, ((0, NPAD - N), (0, 0)))
  w2_p = jnp.pad(W2, ((0, 0), (0, 16 - OUT)))
  b2_p = jnp.pad(b2, (0, 16 - OUT))

  deg2 = _sc_degree(dst_p)
  g = _tc_prep(x_p, feature_importance, W1, deg2)
  parts = _sc_scatter128(g, src_p, dst_p)
  q16 = _tc_mid(parts, g, deg2, b1, gamma, beta, running_mean, running_var,
                w2_p)
  s16 = _sc_scatter16(q16, src_p, dst_p)
  out16 = _tc_final(s16, q16, deg2, b2_p)
  return out16[:N, :OUT]


# speedup vs baseline: 33.0326x; 1.9787x over previous
import functools

import jax
import jax.numpy as jnp
from jax import lax
from jax.experimental import pallas as pl
from jax.experimental.pallas import tpu as pltpu
from jax.experimental.pallas import tpu_sc as plsc

N = 10000
E = 320000
D = 128
OUT = 2

NC = 2
NS = 16
L = 16
NW = NC * NS
NPAD = 10240
E_PAD = 327680
E_PW = E_PAD // NW
CH = 128
NCHUNK = E_PW // CH
ROWS_PER_SUB = NPAD // NS

_mesh = plsc.VectorSubcoreMesh(core_axis_name="c", subcore_axis_name="s")


def _zero_buf(buf, width):
  @pl.loop(0, CH)
  def _(i):
    @pl.loop(0, width, step=L)
    def _(j):
      buf[i, pl.ds(j, L)] = jnp.zeros((L,), jnp.float32)


def _zero_acc_slice(buf, acc, s):
  @pl.loop(0, ROWS_PER_SUB, step=CH)
  def _(r):
    pltpu.sync_copy(buf, acc.at[pl.ds(s * ROWS_PER_SUB + r, CH)])


def _copy_out(acc, out_hbm, c, s):
  pltpu.sync_copy(acc.at[pl.ds(s * ROWS_PER_SUB, ROWS_PER_SUB)],
                  out_hbm.at[c].at[pl.ds(s * ROWS_PER_SUB, ROWS_PER_SUB)])


def _make_sc_ring128():
  width = D
  NB = 2
  NI = 4

  @functools.partial(
      pl.kernel,
      out_type=jax.ShapeDtypeStruct((NC, NPAD, width), jnp.float32),
      mesh=_mesh,
      compiler_params=pltpu.CompilerParams(use_tc_tiling_on_sc=True),
      scratch_types=(
          [pltpu.VMEM((CH, width), jnp.float32)] * NB +
          [pltpu.VMEM((CH,), jnp.int32)] * NI +
          [pltpu.VMEM((CH,), jnp.int32)] * NI +
          [pltpu.VMEM_SHARED((NPAD, width), jnp.float32)] +
          [pltpu.SemaphoreType.DMA] * (2 * NB + NI)
      ),
  )
  def k(table_hbm, src_hbm, dst_hbm, out_hbm, *scr):
    bufs = scr[0:NB]
    srcb = scr[NB:NB + NI]
    dstb = scr[NB + NI:NB + 2 * NI]
    acc = scr[NB + 2 * NI]
    gsem = scr[NB + 2 * NI + 1:NB + 2 * NI + 1 + NB]
    ssem = scr[NB + 2 * NI + 1 + NB:NB + 2 * NI + 1 + 2 * NB]
    isem = scr[NB + 2 * NI + 1 + 2 * NB:]
    c = lax.axis_index("c")
    s = lax.axis_index("s")
    w = c * NS + s

    def start_idx(i, chunk):
      pltpu.async_copy(src_hbm.at[w, chunk], srcb[i], isem[i])
      pltpu.async_copy(dst_hbm.at[w, chunk], dstb[i], isem[i])

    def wait_idx(i, chunk):
      pltpu.make_async_copy(src_hbm.at[w, chunk], srcb[i], isem[i]).wait()
      pltpu.make_async_copy(dst_hbm.at[w, chunk], dstb[i], isem[i]).wait()

    def start_gather(b, i):
      pltpu.async_copy(table_hbm.at[srcb[i]], bufs[b], gsem[b])

    def wait_gather(b, i):
      pltpu.make_async_copy(table_hbm.at[srcb[i]], bufs[b], gsem[b]).wait()

    def start_scatter(b, i):
      pltpu.async_copy(bufs[b], acc.at[dstb[i]], ssem[b], add=True)

    def wait_scatter(b, i):
      pltpu.make_async_copy(bufs[b], acc.at[dstb[i]], ssem[b]).wait()

    _zero_buf(bufs[0], width)
    _zero_acc_slice(bufs[0], acc, s)
    plsc.subcore_barrier()

    for i in range(NI):
      start_idx(i, i)
    wait_idx(0, 0)
    start_gather(0, 0)
    wait_idx(1, 1)
    start_gather(1, 1)

    @pl.loop(0, NCHUNK // NI)
    def _(r):
      c0 = r * NI
      for b in range(NI):
        ci = c0 + b
        rb = b % NB
        wait_gather(rb, b)
        start_scatter(rb, b)
        prb = (b - 1) % NB
        pi = (b - 1) % NI
        gi = (b + 1) % NI

        def _advance():
          wait_scatter(prb, pi)

          @pl.when(ci + 3 < NCHUNK)
          def _():
            start_idx(pi, ci + 3)

          @pl.when(ci + 1 < NCHUNK)
          def _():
            wait_idx(gi, ci + 1)
            start_gather(prb, gi)

        if b > 0:
          _advance()
        else:
          @pl.when(r > 0)
          def _():
            _advance()

    wait_scatter((NCHUNK - 1) % NB, (NCHUNK - 1) % NI)
    plsc.subcore_barrier()
    _copy_out(acc, out_hbm, c, s)

  return k


def _make_sc_scatter(width, nbuf):
  assert NCHUNK % nbuf == 0

  @functools.partial(
      pl.kernel,
      out_type=jax.ShapeDtypeStruct((NC, NPAD, width), jnp.float32),
      mesh=_mesh,
      compiler_params=pltpu.CompilerParams(use_tc_tiling_on_sc=(width == D)),
      scratch_types=(
          [pltpu.VMEM((NCHUNK, CH), jnp.int32)] * 2 +
          [pltpu.VMEM((CH, width), jnp.float32)] * nbuf +
          [pltpu.VMEM_SHARED((NPAD, width), jnp.float32)] +
          [pltpu.SemaphoreType.DMA] * (2 * nbuf)
      ),
  )
  def k(table_hbm, src_hbm, dst_hbm, out_hbm, *scr):
    src_v, dst_v = scr[0], scr[1]
    bufs = scr[2:2 + nbuf]
    acc = scr[2 + nbuf]
    gsem = scr[3 + nbuf:3 + 2 * nbuf]
    ssem = scr[3 + 2 * nbuf:3 + 3 * nbuf]
    c = lax.axis_index("c")
    s = lax.axis_index("s")
    w = c * NS + s

    pltpu.sync_copy(src_hbm.at[w], src_v)
    pltpu.sync_copy(dst_hbm.at[w], dst_v)

    _zero_buf(bufs[0], width)
    _zero_acc_slice(bufs[0], acc, s)
    plsc.subcore_barrier()

    def start_gather(b, chunk):
      pltpu.async_copy(table_hbm.at[src_v.at[chunk]], bufs[b], gsem[b])

    def wait_gather(b, chunk):
      pltpu.make_async_copy(table_hbm.at[src_v.at[chunk]], bufs[b],
                            gsem[b]).wait()

    def start_scatter(b, chunk):
      pltpu.async_copy(bufs[b], acc.at[dst_v.at[chunk]], ssem[b], add=True)

    def wait_scatter(b, chunk):
      pltpu.make_async_copy(bufs[b], acc.at[dst_v.at[chunk]], ssem[b]).wait()

    for b in range(nbuf):
      start_gather(b, b)

    @pl.loop(0, NCHUNK // nbuf)
    def _(r):
      c0 = r * nbuf
      for b in range(nbuf):
        ci = c0 + b
        wait_gather(b, ci)
        start_scatter(b, ci)
        if b > 0:
          wait_scatter(b - 1, ci - 1)
          nxt = ci - 1 + nbuf

          @pl.when(nxt < NCHUNK)
          def _():
            start_gather(b - 1, nxt)
        else:
          @pl.when(r > 0)
          def _():
            wait_scatter(nbuf - 1, ci - 1)
            start_gather(nbuf - 1, ci - 1 + nbuf)

    wait_scatter(nbuf - 1, NCHUNK - 1)
    plsc.subcore_barrier()
    _copy_out(acc, out_hbm, c, s)

  return k


_sc_scatter128 = _make_sc_ring128()
_sc_scatter16 = _make_sc_scatter(16, 8)


@functools.partial(
    pl.kernel,
    out_type=jax.ShapeDtypeStruct((NC, NPAD, 16), jnp.float32),
    mesh=_mesh,
    scratch_types=[
        pltpu.VMEM((NCHUNK, CH), jnp.int32),
        pltpu.VMEM((CH, 16), jnp.float32),
        pltpu.VMEM((CH, 16), jnp.float32),
        pltpu.VMEM_SHARED((NPAD, 16), jnp.float32),
        pltpu.SemaphoreType.DMA,
    ],
)
def _sc_degree(dst_hbm, out_hbm, dst_v, one_v, z_v, acc, sem):
  c = lax.axis_index("c")
  s = lax.axis_index("s")
  w = c * NS + s

  pltpu.sync_copy(dst_hbm.at[w], dst_v)

  @pl.loop(0, CH)
  def _(i):
    one_v[i, :] = jnp.ones((16,), jnp.float32)
    z_v[i, :] = jnp.zeros((16,), jnp.float32)

  _zero_acc_slice(z_v, acc, s)
  plsc.subcore_barrier()

  @pl.loop(0, NCHUNK)
  def _(j):
    pltpu.async_copy(one_v, acc.at[dst_v.at[j]], sem, add=True)

  @pl.loop(0, NCHUNK)
  def _(j):
    pltpu.make_async_copy(one_v, acc.at[dst_v.at[j]], sem).wait()

  plsc.subcore_barrier()
  _copy_out(acc, out_hbm, c, s)


def _tc_prep_body(x_ref, fi_ref, w1_ref, d_ref, g_ref):
  fw = jax.nn.sigmoid(fi_ref[:])
  xw = x_ref[:] * fw[None, :]
  h0 = jnp.dot(xw, w1_ref[:], preferred_element_type=jnp.float32)
  deg = d_ref[0][:, 0:1] + d_ref[1][:, 0:1] + 1.0
  dis = lax.rsqrt(deg)
  g_ref[:] = h0 * dis


def _tc_mid_body(p_ref, g_ref, d_ref, b1_ref, ga_ref, be_ref, rm_ref, rv_ref,
                 w2_ref, q_ref):
  deg = d_ref[0][:, 0:1] + d_ref[1][:, 0:1] + 1.0
  dis = lax.rsqrt(deg)
  h1 = (p_ref[0] + p_ref[1] + g_ref[:]) * dis + b1_ref[:][None, :]
  inv = lax.rsqrt(rv_ref[:] + 1e-5)
  h = (h1 - rm_ref[:][None, :]) * (inv * ga_ref[:])[None, :] + be_ref[:][None, :]
  h = jnp.maximum(h, 0.0)
  q_ref[:] = jnp.dot(h, w2_ref[:], preferred_element_type=jnp.float32) * dis


def _tc_final_body(s_ref, q_ref, d_ref, b2_ref, o_ref):
  deg = d_ref[0][:, 0:1] + d_ref[1][:, 0:1] + 1.0
  dis = lax.rsqrt(deg)
  o_ref[:] = (s_ref[0] + s_ref[1] + q_ref[:]) * dis + b2_ref[:][None, :]


_tc_prep = pl.pallas_call(
    _tc_prep_body, out_shape=jax.ShapeDtypeStruct((NPAD, D), jnp.float32))
_tc_mid = pl.pallas_call(
    _tc_mid_body, out_shape=jax.ShapeDtypeStruct((NPAD, 16), jnp.float32))
_tc_final = pl.pallas_call(
    _tc_final_body, out_shape=jax.ShapeDtypeStruct((NPAD, 16), jnp.float32))


@jax.jit
def kernel(x, edge_index, feature_importance, W1, b1, gamma, beta,
           running_mean, running_var, W2, b2):
  src = edge_index[0].astype(jnp.int32)
  dst = edge_index[1].astype(jnp.int32)
  npadedge = E_PAD - E
  pad_idx = (N + 224 + (jnp.arange(npadedge, dtype=jnp.int32) % 16))
  src_p = jnp.concatenate([src, pad_idx]).reshape(NW, NCHUNK, CH)
  dst_p = jnp.concatenate([dst, pad_idx]).reshape(NW, NCHUNK, CH)
  x_p = jnp.pad(x, ((0, NPAD - N), (0, 0)))
  w2_p = jnp.pad(W2, ((0, 0), (0, 16 - OUT)))
  b2_p = jnp.pad(b2, (0, 16 - OUT))

  deg2 = _sc_degree(dst_p)
  g = _tc_prep(x_p, feature_importance, W1, deg2)
  parts = _sc_scatter128(g, src_p, dst_p)
  q16 = _tc_mid(parts, g, deg2, b1, gamma, beta, running_mean, running_var,
                w2_p)
  s16 = _sc_scatter16(q16, src_p, dst_p)
  out16 = _tc_final(s16, q16, deg2, b2_p)
  return out16[:N, :OUT]
